# Initial kernel scaffold; baseline (speedup 1.0000x reference)
import jax
import jax.numpy as jnp
from jax.experimental import pallas as pl


def kernel(x, selected_experts, routing_weights, W_embed, b_embed, W_proj, b_proj):
    raise NotImplementedError("write your pallas kernel here")


# speedup vs baseline: 1.0467x; 1.0467x over previous
import functools

import jax
import jax.numpy as jnp
from jax.experimental import pallas as pl
from jax.experimental.pallas import tpu as pltpu

IMG = 336
P = 14
G = IMG // P
T = G * G
H = 1024
E = 2
K = 2
PD = 3 * P * P


def _patchify(x):
    b = x.shape[0]
    x = x.reshape(b, 3, G, P, G, P)
    x = jnp.transpose(x, (0, 2, 4, 1, 3, 5))
    return x.reshape(b, T, PD)


def _prep_body(We_ref, be_ref, Wp_ref, bp_ref, M_ref, c_ref):
    M_ref[0] = jnp.dot(We_ref[0], Wp_ref[0], preferred_element_type=jnp.float32)
    c_ref[0] = (
        jnp.dot(be_ref[0], Wp_ref[0], preferred_element_type=jnp.float32)
        + bp_ref[0]
    )


def _moe_body(sel_ref, rw_ref, p_ref, M_ref, c_ref, o_ref):
    b = pl.program_id(0)
    w = []
    for e in range(E):
        acc = jnp.float32(0.0)
        for k in range(K):
            acc += rw_ref[b, k] * (sel_ref[b, k] == e).astype(jnp.float32)
        w.append(acc)
    Mc = w[0] * M_ref[0]
    cc = w[0] * c_ref[0]
    for e in range(1, E):
        Mc += w[e] * M_ref[e]
        cc += w[e] * c_ref[e]
    o_ref[0] = (
        jnp.dot(p_ref[0], Mc, preferred_element_type=jnp.float32) + cc
    )


@functools.partial(jax.jit, static_argnames=())
def kernel(x, selected_experts, routing_weights, W_embed, b_embed, W_proj, b_proj):
    b_sz = x.shape[0]
    patches = _patchify(x)

    M, c = pl.pallas_call(
        _prep_body,
        grid=(E,),
        in_specs=[
            pl.BlockSpec((1, PD, H), lambda e: (e, 0, 0)),
            pl.BlockSpec((1, 1, H), lambda e: (e, 0, 0)),
            pl.BlockSpec((1, H, H), lambda e: (e, 0, 0)),
            pl.BlockSpec((1, 1, H), lambda e: (e, 0, 0)),
        ],
        out_specs=[
            pl.BlockSpec((1, PD, H), lambda e: (e, 0, 0)),
            pl.BlockSpec((1, 1, H), lambda e: (e, 0, 0)),
        ],
        out_shape=[
            jax.ShapeDtypeStruct((E, PD, H), jnp.float32),
            jax.ShapeDtypeStruct((E, 1, H), jnp.float32),
        ],
    )(W_embed, b_embed.reshape(E, 1, H), W_proj, b_proj.reshape(E, 1, H))

    sel = selected_experts.astype(jnp.int32)
    rw = routing_weights.astype(jnp.float32)

    out = pl.pallas_call(
        _moe_body,
        grid=(b_sz,),
        in_specs=[
            pl.BlockSpec(memory_space=pltpu.SMEM),
            pl.BlockSpec(memory_space=pltpu.SMEM),
            pl.BlockSpec((1, T, PD), lambda b: (b, 0, 0)),
            pl.BlockSpec((E, PD, H), lambda b: (0, 0, 0)),
            pl.BlockSpec((E, 1, H), lambda b: (0, 0, 0)),
        ],
        out_specs=pl.BlockSpec((1, T, H), lambda b: (b, 0, 0)),
        out_shape=jax.ShapeDtypeStruct((b_sz, T, H), jnp.float32),
    )(sel, rw, patches, M, c)

    return out
